# baseline (device time: 30997 ns/iter reference)
import jax
import jax.numpy as jnp
from jax import lax
from jax.experimental import pallas as pl
from jax.experimental.pallas import tpu as pltpu

N_CHUNKS = 8


def kernel(x, pi):
    s, m, n = x.shape
    rows = m // N_CHUNKS

    def body(
        x_ref,
        pi_ref,
        out_ref,
        in_stage,
        wire_stage,
        pi_smem,
        in_sems,
        pi_sem,
        send_sems,
        recv_sems,
    ):
        my_x = lax.axis_index("x")
        my_y = lax.axis_index("y")
        other_y = 1 - my_y

        pi_copy = pltpu.make_async_copy(pi_ref, pi_smem, pi_sem)
        pi_copy.start()
        in_copies = []
        for c in range(N_CHUNKS):
            sl = pl.ds(c * rows, rows)
            cp = pltpu.make_async_copy(
                x_ref.at[0, sl], in_stage.at[c % 2], in_sems.at[c % 2]
            )
            in_copies.append(cp)
        in_copies[0].start()
        in_copies[1].start()

        barrier_sem = pltpu.get_barrier_semaphore()
        pl.semaphore_signal(
            barrier_sem,
            inc=1,
            device_id=(my_x, other_y),
            device_id_type=pl.DeviceIdType.MESH,
        )
        pl.semaphore_wait(barrier_sem, 1)

        pi_copy.wait()
        dst_y = pi_smem[my_y]

        rdmas = []
        for c in range(N_CHUNKS):
            sl = pl.ds(c * rows, rows)
            in_copies[c].wait()
            wire_stage[sl, :] = in_stage[c % 2].astype(jnp.bfloat16)
            if c + 2 < N_CHUNKS:
                in_copies[c + 2].start()
            rdma = pltpu.make_async_remote_copy(
                src_ref=wire_stage.at[sl],
                dst_ref=out_ref.at[0, sl],
                send_sem=send_sems.at[c],
                recv_sem=recv_sems.at[c],
                device_id=(my_x, dst_y),
                device_id_type=pl.DeviceIdType.MESH,
            )
            rdma.start()
            rdmas.append(rdma)

        for c in range(N_CHUNKS):
            rdmas[c].wait_recv()
        for c in range(N_CHUNKS):
            rdmas[c].wait_send()

    return pl.pallas_call(
        body,
        out_shape=jax.ShapeDtypeStruct((s, m, n), jnp.bfloat16),
        in_specs=[
            pl.BlockSpec(memory_space=pl.ANY),
            pl.BlockSpec(memory_space=pl.ANY),
        ],
        out_specs=pl.BlockSpec(memory_space=pl.ANY),
        scratch_shapes=[
            pltpu.VMEM((2, rows, n), jnp.float32),
            pltpu.VMEM((m, n), jnp.bfloat16),
            pltpu.SMEM((2,), jnp.int32),
            pltpu.SemaphoreType.DMA((2,)),
            pltpu.SemaphoreType.DMA,
            pltpu.SemaphoreType.DMA((N_CHUNKS,)),
            pltpu.SemaphoreType.DMA((N_CHUNKS,)),
        ],
        compiler_params=pltpu.CompilerParams(collective_id=0),
    )(x, pi)


# device time: 28639 ns/iter; 1.0823x vs baseline; 1.0823x over previous
import jax
import jax.numpy as jnp
from jax import lax
from jax.experimental import pallas as pl
from jax.experimental.pallas import tpu as pltpu

N_CHUNKS = 8


def kernel(x, pi):
    s, m, n = x.shape
    rows = m // N_CHUNKS
    x = pltpu.with_memory_space_constraint(x, pltpu.MemorySpace.HBM)
    pi = pltpu.with_memory_space_constraint(pi, pltpu.MemorySpace.HBM)

    def body(
        x_ref,
        pi_ref,
        out_ref,
        in_stage,
        wire_stage,
        pi_smem,
        in_sems,
        pi_sem,
        send_sems,
        recv_sems,
    ):
        my_x = lax.axis_index("x")
        my_y = lax.axis_index("y")
        other_y = 1 - my_y

        pi_copy = pltpu.make_async_copy(pi_ref, pi_smem, pi_sem)
        pi_copy.start()
        in_copies = []
        for c in range(N_CHUNKS):
            sl = pl.ds(c * rows, rows)
            cp = pltpu.make_async_copy(
                x_ref.at[0, sl], in_stage.at[c % 2], in_sems.at[c % 2]
            )
            in_copies.append(cp)
        in_copies[0].start()
        in_copies[1].start()

        barrier_sem = pltpu.get_barrier_semaphore()
        pl.semaphore_signal(
            barrier_sem,
            inc=1,
            device_id=(my_x, other_y),
            device_id_type=pl.DeviceIdType.MESH,
        )
        pl.semaphore_wait(barrier_sem, 1)

        pi_copy.wait()
        dst_y = pi_smem[my_y]

        rdmas = []
        for c in range(N_CHUNKS):
            sl = pl.ds(c * rows, rows)
            in_copies[c].wait()
            wire_stage[sl, :] = in_stage[c % 2].astype(jnp.bfloat16)
            if c + 2 < N_CHUNKS:
                in_copies[c + 2].start()
            rdma = pltpu.make_async_remote_copy(
                src_ref=wire_stage.at[sl],
                dst_ref=out_ref.at[0, sl],
                send_sem=send_sems.at[c],
                recv_sem=recv_sems.at[c],
                device_id=(my_x, dst_y),
                device_id_type=pl.DeviceIdType.MESH,
            )
            rdma.start()
            rdmas.append(rdma)

        for c in range(N_CHUNKS):
            rdmas[c].wait_recv()
        for c in range(N_CHUNKS):
            rdmas[c].wait_send()

    return pl.pallas_call(
        body,
        out_shape=jax.ShapeDtypeStruct((s, m, n), jnp.bfloat16),
        in_specs=[
            pl.BlockSpec(memory_space=pltpu.MemorySpace.HBM),
            pl.BlockSpec(memory_space=pltpu.MemorySpace.HBM),
        ],
        out_specs=pl.BlockSpec(memory_space=pltpu.MemorySpace.HBM),
        scratch_shapes=[
            pltpu.VMEM((2, rows, n), jnp.float32),
            pltpu.VMEM((m, n), jnp.bfloat16),
            pltpu.SMEM((2,), jnp.int32),
            pltpu.SemaphoreType.DMA((2,)),
            pltpu.SemaphoreType.DMA,
            pltpu.SemaphoreType.DMA((N_CHUNKS,)),
            pltpu.SemaphoreType.DMA((N_CHUNKS,)),
        ],
        compiler_params=pltpu.CompilerParams(collective_id=0),
    )(x, pi)
